# baseline (speedup 1.0000x reference)
import functools

import jax
import jax.numpy as jnp
from jax import lax
from jax.experimental import pallas as pl
from jax.experimental.pallas import tpu as pltpu
from jax.experimental.pallas import tpu_sc as plsc

EMBED_DIM = 64
BATCH = 4096
HIST = 200
B_TOTAL = BATCH * HIST
REPACK_W = 512
REPACK_GRID = 977
SPLIT = REPACK_GRID * REPACK_W

_info = plsc.get_sparse_core_info()
NUM_CORES = _info.num_cores
NUM_SUBCORES = _info.num_subcores
NW = NUM_CORES * NUM_SUBCORES

CHUNK = 512
STEPS = B_TOTAL // (NW * CHUNK)

_mesh = plsc.VectorSubcoreMesh(core_axis_name="c", subcore_axis_name="s")


def _repack_body(lo_ref, hi_ref, y_ref):
  lo = jnp.transpose(lo_ref[...], (1, 0))
  hi = jnp.transpose(hi_ref[...], (1, 0))
  y_ref[...] = jnp.concatenate([lo, hi], axis=1)


_repack = pl.pallas_call(
    _repack_body,
    out_shape=jax.ShapeDtypeStruct((SPLIT, 2 * EMBED_DIM), jnp.float32),
    grid=(REPACK_GRID,),
    in_specs=[
        pl.BlockSpec((EMBED_DIM, REPACK_W), lambda j: (0, j)),
        pl.BlockSpec((EMBED_DIM, REPACK_W), lambda j: (0, j + REPACK_GRID)),
    ],
    out_specs=pl.BlockSpec((REPACK_W, 2 * EMBED_DIM), lambda j: (j, 0)),
    compiler_params=pltpu.CompilerParams(
        dimension_semantics=("parallel",),
    ),
)


@functools.partial(
    pl.kernel,
    mesh=_mesh,
    compiler_params=pltpu.CompilerParams(use_tc_tiling_on_sc=False),
    out_type=jax.ShapeDtypeStruct((B_TOTAL, EMBED_DIM), jnp.float32),
    scratch_types=[
        pltpu.VMEM((CHUNK,), jnp.int32),
        pltpu.VMEM((CHUNK,), jnp.int32),
        pltpu.VMEM((CHUNK, EMBED_DIM), jnp.float32),
        pltpu.SemaphoreType.DMA,
    ],
)
def _gather(y_hbm, idx_hbm, out_hbm, idx_v, midx_v, rows_v, sem):
  wid = lax.axis_index("s") * NUM_CORES + lax.axis_index("c")
  base = wid * (STEPS * CHUNK)

  def chunk(t, carry):
    off = base + t * CHUNK
    pltpu.sync_copy(idx_hbm.at[pl.ds(off, CHUNK)], idx_v)
    for k0 in range(CHUNK // 16):
      iv = idx_v[pl.ds(k0 * 16, 16)]
      midx_v[pl.ds(k0 * 16, 16)] = jnp.where(
          iv < SPLIT, iv * 2, (iv - SPLIT) * 2 + 1)
    pltpu.async_copy(y_hbm.at[midx_v], rows_v, sem).wait()
    pltpu.sync_copy(rows_v, out_hbm.at[pl.ds(off, CHUNK)])
    return carry

  lax.fori_loop(0, STEPS, chunk, 0)


def kernel(x, table):
  tt = table.T
  y = _repack(tt, tt)
  y_rows = y.reshape(2 * SPLIT, EMBED_DIM)
  idx = x.reshape(-1).astype(jnp.int32)
  out = _gather(y_rows, idx)
  return out.reshape(x.shape + (EMBED_DIM,))


# speedup vs baseline: 1.5680x; 1.5680x over previous
import functools

import jax
import jax.numpy as jnp
from jax import lax
from jax.experimental import pallas as pl
from jax.experimental.pallas import tpu as pltpu
from jax.experimental.pallas import tpu_sc as plsc

EMBED_DIM = 64
BATCH = 4096
HIST = 200
B_TOTAL = BATCH * HIST
VOCAB1 = 1000001
REPACK_W = 4096
XPOSE_W = 512
REPACK_GRID = 245
Y_ROWS = REPACK_GRID * REPACK_W

_info = plsc.get_sparse_core_info()
NUM_CORES = _info.num_cores
NUM_SUBCORES = _info.num_subcores
NW = NUM_CORES * NUM_SUBCORES

CHUNK = 256
STEPS = B_TOTAL // (NW * CHUNK)

_mesh = plsc.VectorSubcoreMesh(core_axis_name="c", subcore_axis_name="s")


def _repack_body(tt_ref, y_ref):
  v = tt_ref[...]
  parts = []
  for k in range(REPACK_W // XPOSE_W):
    sl = lax.slice(v, (0, k * XPOSE_W), (EMBED_DIM, (k + 1) * XPOSE_W))
    parts.append(jnp.transpose(sl, (1, 0)))
  t = jnp.concatenate(parts, axis=0)
  y_ref[...] = jnp.concatenate([t, t], axis=1)


_repack = pl.pallas_call(
    _repack_body,
    out_shape=jax.ShapeDtypeStruct((Y_ROWS, 2 * EMBED_DIM), jnp.float32),
    grid=(REPACK_GRID,),
    in_specs=[pl.BlockSpec((EMBED_DIM, REPACK_W), lambda j: (0, j))],
    out_specs=pl.BlockSpec((REPACK_W, 2 * EMBED_DIM), lambda j: (j, 0)),
    compiler_params=pltpu.CompilerParams(
        dimension_semantics=("parallel",),
    ),
)


@functools.partial(
    pl.kernel,
    mesh=_mesh,
    out_type=jax.ShapeDtypeStruct((B_TOTAL, EMBED_DIM), jnp.float32),
    scratch_types=[
        pltpu.VMEM((CHUNK,), jnp.int32),
        pltpu.VMEM((CHUNK,), jnp.int32),
        pltpu.VMEM((CHUNK, 2 * EMBED_DIM), jnp.float32),
        pltpu.VMEM((CHUNK, 2 * EMBED_DIM), jnp.float32),
        pltpu.VMEM((CHUNK, EMBED_DIM), jnp.float32),
        pltpu.SemaphoreType.DMA,
        pltpu.SemaphoreType.DMA,
    ],
)
def _gather(y_hbm, idx_hbm, out_hbm,
            idx0_v, idx1_v, rows0_v, rows1_v, sel_v, sem0, sem1):
  wid = lax.axis_index("s") * NUM_CORES + lax.axis_index("c")
  base = wid * (STEPS * CHUNK)

  idx_bufs = (idx0_v, idx1_v)
  rows_bufs = (rows0_v, rows1_v)
  sems = (sem0, sem1)

  def select(rows_v):
    def body(k4, carry):
      for u in range(4):
        k = k4 * 4 + u
        for d0 in range(0, EMBED_DIM, 16):
          sel_v[k, pl.ds(d0, 16)] = rows_v[k, pl.ds(d0, 16)]
      return carry

    lax.fori_loop(0, CHUNK // 4, body, 0)

  pltpu.sync_copy(idx_hbm.at[pl.ds(base, CHUNK)], idx_bufs[0])
  pltpu.async_copy(y_hbm.at[idx_bufs[0]], rows_bufs[0], sems[0])

  def chunk(t, carry):
    cur = lax.rem(t, 2)
    nxt = 1 - cur

    @pl.when(t + 1 < STEPS)
    def _():
      for b in range(2):
        @pl.when(nxt == b)
        def _():
          pltpu.sync_copy(idx_hbm.at[pl.ds(base + (t + 1) * CHUNK, CHUNK)],
                          idx_bufs[b])
          pltpu.async_copy(y_hbm.at[idx_bufs[b]], rows_bufs[b], sems[b])

    for b in range(2):
      @pl.when(cur == b)
      def _():
        pltpu.make_async_copy(y_hbm.at[idx_bufs[b]], rows_bufs[b],
                              sems[b]).wait()
        select(rows_bufs[b])
    pltpu.sync_copy(sel_v, out_hbm.at[pl.ds(base + t * CHUNK, CHUNK), :])
    return carry

  lax.fori_loop(0, STEPS, chunk, 0)


def kernel(x, table):
  y = _repack(table.T)
  idx = x.reshape(-1).astype(jnp.int32)
  out = _gather(y, idx)
  return out.reshape(x.shape + (EMBED_DIM,))
